# Initial kernel scaffold; baseline (speedup 1.0000x reference)
import jax
import jax.numpy as jnp
from jax.experimental import pallas as pl


def kernel(x, router_logits, w_gate, w_up, w_down):
    raise NotImplementedError("write your pallas kernel here")


# speedup vs baseline: 1.4278x; 1.4278x over previous
import functools

import jax
import jax.numpy as jnp
from jax import lax
from jax.experimental import pallas as pl
from jax.experimental.pallas import tpu as pltpu

_E = 8
_K = 2


def _routing_body(logits_ref, comb_ref):
    lg = logits_ref[...]
    m = jnp.max(lg, axis=-1, keepdims=True)
    ex = jnp.exp(lg - m)
    p = ex / jnp.sum(ex, axis=-1, keepdims=True)
    lane = lax.broadcasted_iota(jnp.int32, p.shape, 1)
    m1 = jnp.max(p, axis=-1, keepdims=True)
    is1 = p >= m1
    lane1 = jnp.min(jnp.where(is1, lane, _E), axis=-1, keepdims=True)
    mask1 = lane == lane1
    p2 = jnp.where(mask1, -1.0, p)
    m2 = jnp.max(p2, axis=-1, keepdims=True)
    is2 = p2 >= m2
    lane2 = jnp.min(jnp.where(is2, lane, _E), axis=-1, keepdims=True)
    mask2 = lane == lane2
    denom = m1 + m2
    comb_ref[...] = jnp.where(mask1 | mask2, p, 0.0) / denom


def _dense_body(x_ref, wg_ref, wu_ref, wd_ref, comb_ref, out_ref, acc_ref, *, bt):
    e = pl.program_id(0)
    tb = pl.program_id(1)
    xb = x_ref[...].astype(jnp.bfloat16)
    wg = wg_ref[0].astype(jnp.bfloat16)
    wu = wu_ref[0].astype(jnp.bfloat16)
    wd = wd_ref[0].astype(jnp.bfloat16)
    gate = jnp.dot(xb, wg, preferred_element_type=jnp.float32)
    up = jnp.dot(xb, wu, preferred_element_type=jnp.float32)
    h = (gate * jax.nn.sigmoid(gate) * up).astype(jnp.bfloat16)
    y = jnp.dot(h, wd, preferred_element_type=jnp.float32)
    onehot = (lax.broadcasted_iota(jnp.int32, (1, _E), 1) == e).astype(jnp.float32)
    w_col = jnp.sum(comb_ref[...] * onehot, axis=1, keepdims=True)
    contrib = y * w_col
    sl = pl.ds(tb * bt, bt)

    @pl.when(e == 0)
    def _():
        acc_ref[sl, :] = contrib

    @pl.when(e > 0)
    def _():
        acc_ref[sl, :] += contrib

    @pl.when(e == _E - 1)
    def _():
        out_ref[...] = acc_ref[sl, :]


def kernel(x, router_logits, w_gate, w_up, w_down):
    T, H = x.shape
    E, _, F = w_gate.shape
    BT = 512

    comb = pl.pallas_call(
        _routing_body,
        out_shape=jax.ShapeDtypeStruct((T, _E), jnp.float32),
    )(router_logits)

    out = pl.pallas_call(
        functools.partial(_dense_body, bt=BT),
        grid=(E, T // BT),
        in_specs=[
            pl.BlockSpec((BT, H), lambda e, tb: (tb, 0)),
            pl.BlockSpec((1, H, F), lambda e, tb: (e, 0, 0)),
            pl.BlockSpec((1, H, F), lambda e, tb: (e, 0, 0)),
            pl.BlockSpec((1, F, H), lambda e, tb: (e, 0, 0)),
            pl.BlockSpec((BT, _E), lambda e, tb: (tb, 0)),
        ],
        out_specs=pl.BlockSpec((BT, H), lambda e, tb: (tb, 0)),
        out_shape=jax.ShapeDtypeStruct((T, H), jnp.float32),
        scratch_shapes=[pltpu.VMEM((T, H), jnp.float32)],
    )(x, w_gate, w_up, w_down, comb)
    return out
